# baseline (speedup 1.0000x reference)
import jax
import jax.numpy as jnp
from jax import lax
from jax.experimental import pallas as pl
from jax.experimental.pallas import tpu as pltpu
from jax.experimental.pallas import tpu_sc as plsc

E = 512
M = 16
S = 8
H = 768
NW = 32
E_PER_W = E // NW
IDX_PER_W = E_PER_W * M * S
ROWS_HALF = M * S // 2
LANES = 16


def _tree_sum(vals):
    while len(vals) > 1:
        vals = [vals[i] + vals[i + 1] for i in range(0, len(vals) - 1, 2)] + (
            [vals[-1]] if len(vals) % 2 else []
        )
    return vals[0]


def _sc_body(idx_hbm, tok_hbm, men_out, ent_out, idx_v, gbufs, mens, ents, sems):
    semA, semB, semW0, semW1 = sems
    sem_w = (semW0, semW1)
    c = lax.axis_index("c")
    s = lax.axis_index("s")
    wid = s * 2 + c
    base_e = wid * E_PER_W

    pltpu.sync_copy(idx_hbm.at[pl.ds(wid * IDX_PER_W, IDX_PER_W)], idx_v)

    def idx_slice(local_e, half):
        return idx_v.at[pl.ds(local_e * (M * S) + half * ROWS_HALF, ROWS_HALF)]

    def start_gather(local_e, half, buf, sem):
        pltpu.async_copy(tok_hbm.at[idx_slice(local_e, half)], buf, sem)

    def wait_gather(local_e, half, buf, sem):
        pltpu.make_async_copy(tok_hbm.at[idx_slice(local_e, half)], buf, sem).wait()

    def compute_half(buf, men, ent, half):
        @plsc.parallel_loop(0, H, step=LANES, unroll=2)
        def _chunk(cc):
            accs = []
            for m in range(M // 2):
                r0 = m * S
                acc = _tree_sum(
                    [buf[r0 + r, pl.ds(cc, LANES)] for r in range(S)]
                )
                men[half * (M // 2) + m, pl.ds(cc, LANES)] = acc * (1.0 / S)
                accs.append(acc)
            eacc = _tree_sum(accs) * (1.0 / (M * S))
            if half == 0:
                ent[0, pl.ds(cc, LANES)] = eacc
            else:
                plsc.addupdate(ent.at[0, pl.ds(cc, LANES)], eacc)

    start_gather(0, 0, gbufs[0], semA)

    @pl.loop(0, E_PER_W, step=2)
    def _entity_pair(i):
        for ei in range(2):
            local_e = i + ei
            e = base_e + local_e
            men = mens[ei]
            ent = ents[ei]

            start_gather(local_e, 1, gbufs[1], semB)
            wait_gather(local_e, 0, gbufs[0], semA)

            @pl.when(local_e >= 2)
            def _drain():
                pltpu.make_async_copy(
                    men, men_out.at[pl.ds(e * M, M)], sem_w[ei]
                ).wait()
                pltpu.make_async_copy(
                    ent, ent_out.at[pl.ds(e, 1)], sem_w[ei]
                ).wait()

            compute_half(gbufs[0], men, ent, 0)

            start_gather((local_e + 1) & (E_PER_W - 1), 0, gbufs[0], semA)
            wait_gather(local_e, 1, gbufs[1], semB)
            compute_half(gbufs[1], men, ent, 1)

            pltpu.async_copy(men, men_out.at[pl.ds(e * M, M)], sem_w[ei])
            pltpu.async_copy(ent, ent_out.at[pl.ds(e, 1)], sem_w[ei])

    for ei in range(2):
        pltpu.make_async_copy(
            mens[ei], men_out.at[pl.ds(0, M)], sem_w[ei]
        ).wait()
        pltpu.make_async_copy(ents[ei], ent_out.at[pl.ds(0, 1)], sem_w[ei]).wait()


@jax.jit
def _sc_entity_repr(token_repr, idx_flat):
    mesh = plsc.VectorSubcoreMesh(core_axis_name="c", subcore_axis_name="s")
    run = pl.kernel(
        _sc_body,
        out_type=[
            jax.ShapeDtypeStruct((E * M, H), jnp.float32),
            jax.ShapeDtypeStruct((E, H), jnp.float32),
        ],
        mesh=mesh,
        scratch_types=[
            pltpu.VMEM((IDX_PER_W,), jnp.int32),
            [pltpu.VMEM((ROWS_HALF, H), jnp.float32) for _ in range(2)],
            [pltpu.VMEM((M, H), jnp.float32) for _ in range(2)],
            [pltpu.VMEM((1, H), jnp.float32) for _ in range(2)],
            [pltpu.SemaphoreType.DMA for _ in range(4)],
        ],
    )
    return run(idx_flat, token_repr)


def kernel(token_repr, pos_idx):
    idx_flat = pos_idx.astype(jnp.int32).reshape(-1)
    men, ent = _sc_entity_repr(token_repr, idx_flat)
    mentions_reprs = men.reshape(E, M, H)
    mentions_mask = jnp.ones((E, M), dtype=jnp.float32)
    return (ent, mentions_reprs, mentions_mask)


# speedup vs baseline: 1.2580x; 1.2580x over previous
import jax
import jax.numpy as jnp
from jax import lax
from jax.experimental import pallas as pl
from jax.experimental.pallas import tpu as pltpu
from jax.experimental.pallas import tpu_sc as plsc

E = 512
M = 16
S = 8
H = 768
NW = 32
E_PER_W = E // NW
IDX_PER_W = E_PER_W * M * S
ROWS_HALF = M * S // 2
LANES = 16


def _tree_sum(vals):
    while len(vals) > 1:
        vals = [vals[i] + vals[i + 1] for i in range(0, len(vals) - 1, 2)] + (
            [vals[-1]] if len(vals) % 2 else []
        )
    return vals[0]


def _sc_body(idx_hbm, tok_hbm, men_out, ent_out, idx_v, gbufs, mens, ents, sems):
    semA, semB, semW0, semW1 = sems
    sem_w = (semW0, semW1)
    c = lax.axis_index("c")
    s = lax.axis_index("s")
    wid = s * 2 + c
    base_e = wid * E_PER_W

    pltpu.sync_copy(idx_hbm.at[pl.ds(wid * IDX_PER_W, IDX_PER_W)], idx_v)

    def idx_slice(local_e, half):
        return idx_v.at[pl.ds(local_e * (M * S) + half * ROWS_HALF, ROWS_HALF)]

    def start_gather(local_e, half, buf, sem):
        pltpu.async_copy(tok_hbm.at[idx_slice(local_e, half)], buf, sem)

    def wait_gather(local_e, half, buf, sem):
        pltpu.make_async_copy(tok_hbm.at[idx_slice(local_e, half)], buf, sem).wait()

    def compute_half(buf, men, ent, half):
        @plsc.parallel_loop(0, H, step=LANES)
        def _chunk(cc):
            accs = []
            for m in range(M // 2):
                r0 = m * S
                acc = _tree_sum(
                    [buf[r0 + r, pl.ds(cc, LANES)] for r in range(S)]
                )
                men[half * (M // 2) + m, pl.ds(cc, LANES)] = acc * (1.0 / S)
                accs.append(acc)
            eacc = _tree_sum(accs) * (1.0 / (M * S))
            if half == 0:
                ent[0, pl.ds(cc, LANES)] = eacc
            else:
                plsc.addupdate(ent.at[0, pl.ds(cc, LANES)], eacc)

    start_gather(0, 0, gbufs[0], semA)

    @pl.loop(0, E_PER_W, step=2)
    def _entity_pair(i):
        for ei in range(2):
            local_e = i + ei
            e = base_e + local_e
            men = mens[ei]
            ent = ents[ei]

            start_gather(local_e, 1, gbufs[1], semB)
            wait_gather(local_e, 0, gbufs[0], semA)

            @pl.when(local_e >= 2)
            def _drain():
                pltpu.make_async_copy(
                    men, men_out.at[pl.ds(e * M, M)], sem_w[ei]
                ).wait()
                pltpu.make_async_copy(
                    ent, ent_out.at[pl.ds(e, 1)], sem_w[ei]
                ).wait()

            compute_half(gbufs[0], men, ent, 0)

            start_gather((local_e + 1) & (E_PER_W - 1), 0, gbufs[0], semA)
            wait_gather(local_e, 1, gbufs[1], semB)
            compute_half(gbufs[1], men, ent, 1)

            pltpu.async_copy(men, men_out.at[pl.ds(e * M, M)], sem_w[ei])
            pltpu.async_copy(ent, ent_out.at[pl.ds(e, 1)], sem_w[ei])

    for ei in range(2):
        pltpu.make_async_copy(
            mens[ei], men_out.at[pl.ds(0, M)], sem_w[ei]
        ).wait()
        pltpu.make_async_copy(ents[ei], ent_out.at[pl.ds(0, 1)], sem_w[ei]).wait()


@jax.jit
def _sc_entity_repr(token_repr, idx_flat):
    mesh = plsc.VectorSubcoreMesh(core_axis_name="c", subcore_axis_name="s")
    run = pl.kernel(
        _sc_body,
        out_type=[
            jax.ShapeDtypeStruct((E * M, H), jnp.float32),
            jax.ShapeDtypeStruct((E, H), jnp.float32),
        ],
        mesh=mesh,
        scratch_types=[
            pltpu.VMEM((IDX_PER_W,), jnp.int32),
            [pltpu.VMEM((ROWS_HALF, H), jnp.float32) for _ in range(2)],
            [pltpu.VMEM((M, H), jnp.float32) for _ in range(2)],
            [pltpu.VMEM((1, H), jnp.float32) for _ in range(2)],
            [pltpu.SemaphoreType.DMA for _ in range(4)],
        ],
    )
    return run(idx_flat, token_repr)


def kernel(token_repr, pos_idx):
    idx_flat = pos_idx.astype(jnp.int32).reshape(-1)
    men, ent = _sc_entity_repr(token_repr, idx_flat)
    mentions_reprs = men.reshape(E, M, H)
    mentions_mask = jnp.ones((E, M), dtype=jnp.float32)
    return (ent, mentions_reprs, mentions_mask)
